# Initial kernel scaffold; baseline (speedup 1.0000x reference)
import jax
import jax.numpy as jnp
from jax.experimental import pallas as pl


def kernel(x_enc, neighbors, samples_x, samples_y):
    raise NotImplementedError("write your pallas kernel here")


# speedup vs baseline: 3.6150x; 3.6150x over previous
import functools

import jax
import jax.numpy as jnp
from jax import lax
from jax.experimental import pallas as pl
from jax.experimental.pallas import tpu as pltpu
from jax.experimental.pallas import tpu_sc as plsc

B = 1024
NCOMP = 200
D = 128
DY = 16
NPAD = 208
SIGMA = 8.0
EPS = 1e-12

_NC = 2
_NS = 16
_NW = _NC * _NS
_QPW = B // _NW

_CHUNKS = ((0, 104), (104, 96))


def _sc_gather_body(sx_hbm, sy_hbm, nbr_hbm, gx_hbm, gy_hbm,
                    idx_v, rx_v, ry_v, sem):
    wid = lax.axis_index("s") * _NC + lax.axis_index("c")
    base = wid * _QPW

    def step(i, carry):
        q = base + i
        pltpu.sync_copy(nbr_hbm.at[q], idx_v)
        cps = []
        for (o, ln) in _CHUNKS:
            cps.append(pltpu.make_async_copy(
                sx_hbm.at[idx_v.at[pl.ds(o, ln)]], rx_v.at[pl.ds(o, ln)], sem))
            cps.append(pltpu.make_async_copy(
                sy_hbm.at[idx_v.at[pl.ds(o, ln)]], ry_v.at[pl.ds(o, ln)], sem))
        for c in cps:
            c.start()
        for c in cps:
            c.wait()
        pltpu.sync_copy(rx_v, gx_hbm.at[q])
        pltpu.sync_copy(ry_v, gy_hbm.at[q])
        return carry

    lax.fori_loop(0, _QPW, step, 0)


@functools.cache
def _sc_gather():
    return pl.kernel(
        _sc_gather_body,
        mesh=plsc.VectorSubcoreMesh(
            core_axis_name="c", subcore_axis_name="s",
            num_cores=_NC, num_subcores=_NS),
        out_type=[
            jax.ShapeDtypeStruct((B, NCOMP, D), jnp.float32),
            jax.ShapeDtypeStruct((B, NCOMP, DY), jnp.float32),
        ],
        scratch_types=[
            pltpu.VMEM((NPAD,), jnp.int32),
            pltpu.VMEM((NCOMP, D), jnp.float32),
            pltpu.VMEM((NCOMP, DY), jnp.float32),
            pltpu.SemaphoreType.DMA,
        ],
        compiler_params=pltpu.CompilerParams(use_tc_tiling_on_sc=False),
    )


_QBLK = 8


def _tc_body(x_ref, gx_ref, gy_ref, out_ref):
    inv2s2 = -1.0 / (2.0 * SIGMA * SIGMA)
    for q in range(_QBLK):
        rows = gx_ref[q]
        diff = rows - x_ref[q][None, :]
        d2 = jnp.sum(diff * diff, axis=1, keepdims=True)
        k2 = jnp.exp(d2 * (2.0 * inv2s2))
        w = k2 / (jnp.sum(k2) + EPS)
        y = gy_ref[q]
        n2 = jnp.sum(y * y, axis=1, keepdims=True)
        denom = jnp.sqrt(n2) + EPS
        coef = w / (denom * denom)
        out_ref[q, :] = jnp.sum(coef * y * y, axis=0)


def _tc_compute(x_enc, gx, gy):
    return pl.pallas_call(
        _tc_body,
        grid=(B // _QBLK,),
        in_specs=[
            pl.BlockSpec((_QBLK, D), lambda i: (i, 0)),
            pl.BlockSpec((_QBLK, NCOMP, D), lambda i: (i, 0, 0)),
            pl.BlockSpec((_QBLK, NCOMP, DY), lambda i: (i, 0, 0)),
        ],
        out_specs=pl.BlockSpec((_QBLK, DY), lambda i: (i, 0)),
        out_shape=jax.ShapeDtypeStruct((B, DY), jnp.float32),
    )(x_enc, gx, gy)


def kernel(x_enc, neighbors, samples_x, samples_y):
    sy_pad = jnp.pad(samples_y, ((0, 0), (0, DY - samples_y.shape[1])))
    nbr_pad = jnp.pad(neighbors, ((0, 0), (0, NPAD - NCOMP)))
    gx, gy = _sc_gather()(samples_x, sy_pad, nbr_pad)
    out = _tc_compute(x_enc, gx, gy)
    return out[:, :samples_y.shape[1]]
